# baseline (speedup 1.0000x reference)
import functools

import jax
import jax.numpy as jnp
from jax import lax
from jax.experimental import pallas as pl
from jax.experimental.pallas import tpu as pltpu
from jax.experimental.pallas import tpu_sc as plsc

_CAPACITY = 16384
_KEY_BITS = 2048
_VALUE_BITS = 2048

_NC = 2
_NS = 16
_NW = _NC * _NS
_L = 16

_ROWS_PER_W = _CAPACITY // _NW
_G = 8
_GROUPS = _ROWS_PER_W // _G
_CHUNKS = _KEY_BITS // _L


def _sc_scan_body(q_hbm, keys_hbm, comb_hbm, q_v, kbuf, res_v, sem):
    wid = lax.axis_index("s") * _NC + lax.axis_index("c")
    base = wid * _ROWS_PER_W
    pltpu.sync_copy(q_hbm, q_v)

    def group_body(g, best):
        row0 = base + g * _G
        pltpu.async_copy(keys_hbm.at[pl.ds(row0, _G)], kbuf, sem).wait()

        def chunk_body(j, accs):
            off = j * _L
            q = q_v[pl.ds(off, _L)]
            return tuple(
                accs[r]
                + plsc.all_reduce_population_count(kbuf[r, pl.ds(off, _L)] != q)
                for r in range(_G)
            )

        zero = jnp.zeros((_L,), jnp.int32)
        accs = lax.fori_loop(0, _CHUNKS, chunk_body, (zero,) * _G)
        for r in range(_G):
            combined = accs[r] * _CAPACITY + (row0 + r)
            best = jnp.minimum(best, combined)
        return best

    big = jnp.full((_L,), 2**30, jnp.int32)
    best = lax.fori_loop(0, _GROUPS, group_body, big)
    res_v[...] = best
    pltpu.sync_copy(res_v, comb_hbm.at[wid])


def _sc_scan(query, keys):
    kern = pl.kernel(
        _sc_scan_body,
        out_type=jax.ShapeDtypeStruct((_NW, _L), jnp.int32),
        mesh=plsc.VectorSubcoreMesh(core_axis_name="c", subcore_axis_name="s"),
        scratch_types=[
            pltpu.VMEM((_KEY_BITS,), jnp.int32),
            pltpu.VMEM((_G, _KEY_BITS), jnp.int32),
            pltpu.VMEM((_L,), jnp.int32),
            pltpu.SemaphoreType.DMA,
        ],
        compiler_params=pltpu.CompilerParams(needs_layout_passes=False),
    )
    return kern(query, keys)


def _merge_body(comb_ref, values_hbm, out_ref, sem):
    best = jnp.min(comb_ref[...])
    idx = jnp.bitwise_and(best, _CAPACITY - 1)
    copy = pltpu.make_async_copy(values_hbm.at[idx], out_ref, sem)
    copy.start()
    copy.wait()


def _merge_gather(comb, values):
    return pl.pallas_call(
        _merge_body,
        in_specs=[
            pl.BlockSpec(memory_space=pltpu.VMEM),
            pl.BlockSpec(memory_space=pltpu.MemorySpace.HBM),
        ],
        out_specs=pl.BlockSpec(memory_space=pltpu.VMEM),
        out_shape=jax.ShapeDtypeStruct((_VALUE_BITS,), jnp.float32),
        scratch_shapes=[pltpu.SemaphoreType.DMA],
    )(comb, values)


def kernel(query, keys, values):
    comb = _sc_scan(query, keys)
    return _merge_gather(comb, values)


# speedup vs baseline: 1.8414x; 1.8414x over previous
import functools

import jax
import jax.numpy as jnp
from jax import lax
from jax.experimental import pallas as pl
from jax.experimental.pallas import tpu as pltpu
from jax.experimental.pallas import tpu_sc as plsc

_CAPACITY = 16384
_KEY_BITS = 2048
_VALUE_BITS = 2048

_NC = 2
_NS = 16
_NW = _NC * _NS
_L = 16

_ROWS_PER_W = _CAPACITY // _NW
_G = 16
_GROUPS = _ROWS_PER_W // _G
_CHUNKS = _KEY_BITS // _L
_UNROLL = 2


def _sc_group_start(keys_hbm, kbuf, sems, base, g, b):
    copy = pltpu.make_async_copy(
        keys_hbm.at[pl.ds(base + g * _G, _G)], kbuf.at[b], sems[b]
    )
    copy.start()


def _sc_scan_body(q_hbm, keys_hbm, comb_hbm, q_v, kbuf, res_v, sem0, sem1):
    sems = (sem0, sem1)
    wid = lax.axis_index("s") * _NC + lax.axis_index("c")
    base = wid * _ROWS_PER_W
    pltpu.sync_copy(q_hbm, q_v)
    _sc_group_start(keys_hbm, kbuf, sems, base, 0, 0)

    def pair_body(p, best):
        for b in range(2):
            g = p * 2 + b

            @pl.when(g + 1 < _GROUPS)
            def _start_next():
                _sc_group_start(keys_hbm, kbuf, sems, base, g + 1, 1 - b)

            pltpu.make_async_copy(
                keys_hbm.at[pl.ds(base + g * _G, _G)], kbuf.at[b], sems[b]
            ).wait()

            def chunk_body(j, accs):
                for u in range(_UNROLL):
                    off = (j * _UNROLL + u) * _L
                    q = q_v[pl.ds(off, _L)]
                    accs = tuple(
                        accs[r]
                        + plsc.all_reduce_population_count(
                            kbuf[b, r, pl.ds(off, _L)] != q
                        )
                        for r in range(_G)
                    )
                return accs

            zero = jnp.zeros((_L,), jnp.int32)
            accs = lax.fori_loop(0, _CHUNKS // _UNROLL, chunk_body, (zero,) * _G)
            row0 = base + g * _G
            for r in range(_G):
                combined = accs[r] * _CAPACITY + (row0 + r)
                best = jnp.minimum(best, combined)
        return best

    big = jnp.full((_L,), 2**30, jnp.int32)
    best = lax.fori_loop(0, _GROUPS // 2, pair_body, big)
    res_v[...] = best
    pltpu.sync_copy(res_v, comb_hbm.at[wid])


def _sc_scan(query, keys):
    kern = pl.kernel(
        _sc_scan_body,
        out_type=jax.ShapeDtypeStruct((_NW, _L), jnp.int32),
        mesh=plsc.VectorSubcoreMesh(core_axis_name="c", subcore_axis_name="s"),
        scratch_types=[
            pltpu.VMEM((_KEY_BITS,), jnp.int32),
            pltpu.VMEM((2, _G, _KEY_BITS), jnp.int32),
            pltpu.VMEM((_L,), jnp.int32),
            pltpu.SemaphoreType.DMA,
            pltpu.SemaphoreType.DMA,
        ],
        compiler_params=pltpu.CompilerParams(needs_layout_passes=False),
    )
    return kern(query, keys)


def _merge_body(comb_ref, values_hbm, out_ref, sem):
    best = jnp.min(comb_ref[...])
    idx = jnp.bitwise_and(best, _CAPACITY - 1)
    copy = pltpu.make_async_copy(values_hbm.at[idx], out_ref, sem)
    copy.start()
    copy.wait()


def _merge_gather(comb, values):
    return pl.pallas_call(
        _merge_body,
        in_specs=[
            pl.BlockSpec(memory_space=pltpu.VMEM),
            pl.BlockSpec(memory_space=pltpu.MemorySpace.HBM),
        ],
        out_specs=pl.BlockSpec(memory_space=pltpu.VMEM),
        out_shape=jax.ShapeDtypeStruct((_VALUE_BITS,), jnp.float32),
        scratch_shapes=[pltpu.SemaphoreType.DMA],
    )(comb, values)


def kernel(query, keys, values):
    comb = _sc_scan(query, keys)
    return _merge_gather(comb, values)


# speedup vs baseline: 2.3556x; 1.2793x over previous
import jax
import jax.numpy as jnp
from jax import lax
from jax.experimental import pallas as pl
from jax.experimental.pallas import tpu as pltpu
from jax.experimental.pallas import tpu_sc as plsc

_CAPACITY = 16384
_KEY_BITS = 2048
_VALUE_BITS = 2048

_SC_ROWS = 6144
_TC_ROWS = _CAPACITY - _SC_ROWS

_NC = 2
_NS = 16
_NW = _NC * _NS
_L = 16

_ROWS_PER_W = _SC_ROWS // _NW
_G = 16
_GROUPS = _ROWS_PER_W // _G
_CHUNKS = _KEY_BITS // _L
_UNROLL = 2

_BLK = 1024


def _sc_group_start(keys_hbm, kbuf, sems, base, g, b):
    copy = pltpu.make_async_copy(
        keys_hbm.at[pl.ds(base + g * _G, _G)], kbuf.at[b], sems[b]
    )
    copy.start()


def _sc_scan_body(q_hbm, keys_hbm, comb_hbm, q_v, kbuf, res_v, sem0, sem1):
    sems = (sem0, sem1)
    wid = lax.axis_index("s") * _NC + lax.axis_index("c")
    base = wid * _ROWS_PER_W
    pltpu.sync_copy(q_hbm, q_v)
    _sc_group_start(keys_hbm, kbuf, sems, base, 0, 0)

    def pair_body(p, best):
        for b in range(2):
            g = p * 2 + b

            @pl.when(g + 1 < _GROUPS)
            def _start_next():
                _sc_group_start(keys_hbm, kbuf, sems, base, g + 1, 1 - b)

            pltpu.make_async_copy(
                keys_hbm.at[pl.ds(base + g * _G, _G)], kbuf.at[b], sems[b]
            ).wait()

            def chunk_body(j, accs):
                for u in range(_UNROLL):
                    off = (j * _UNROLL + u) * _L
                    q = q_v[pl.ds(off, _L)]
                    accs = tuple(
                        accs[r]
                        + plsc.all_reduce_population_count(
                            kbuf[b, r, pl.ds(off, _L)] != q
                        )
                        for r in range(_G)
                    )
                return accs

            zero = jnp.zeros((_L,), jnp.int32)
            accs = lax.fori_loop(0, _CHUNKS // _UNROLL, chunk_body, (zero,) * _G)
            row0 = base + g * _G
            for r in range(_G):
                combined = accs[r] * _CAPACITY + (row0 + r)
                best = jnp.minimum(best, combined)
        return best

    big = jnp.full((_L,), 2**30, jnp.int32)
    best = lax.fori_loop(0, _GROUPS // 2, pair_body, big)
    res_v[...] = best
    pltpu.sync_copy(res_v, comb_hbm.at[wid])


def _sc_scan(query, keys):
    kern = pl.kernel(
        _sc_scan_body,
        out_type=jax.ShapeDtypeStruct((_NW, _L), jnp.int32),
        mesh=plsc.VectorSubcoreMesh(core_axis_name="c", subcore_axis_name="s"),
        scratch_types=[
            pltpu.VMEM((_KEY_BITS,), jnp.int32),
            pltpu.VMEM((2, _G, _KEY_BITS), jnp.int32),
            pltpu.VMEM((_L,), jnp.int32),
            pltpu.SemaphoreType.DMA,
            pltpu.SemaphoreType.DMA,
        ],
        compiler_params=pltpu.CompilerParams(needs_layout_passes=False),
    )
    return kern(query, keys)


def _tc_scan_body(q_ref, keys_ref, out_ref, best_ref):
    i = pl.program_id(0)
    nblk = pl.num_programs(0)

    @pl.when(i == 0)
    def _init():
        best_ref[0] = jnp.int32(2**30)

    xor = jnp.bitwise_xor(keys_ref[...], q_ref[...])
    dist = jnp.sum(xor, axis=1, keepdims=True)
    rows = lax.broadcasted_iota(jnp.int32, dist.shape, 0)
    combined = dist * _CAPACITY + (_SC_ROWS + i * _BLK + rows)
    blk_best = jnp.min(combined)
    best_ref[0] = jnp.minimum(best_ref[0], blk_best)

    @pl.when(i == nblk - 1)
    def _emit():
        out_ref[0] = best_ref[0]


def _tc_scan(query, keys):
    q2 = query.reshape(1, _KEY_BITS)
    grid = _TC_ROWS // _BLK
    sc_blocks = _SC_ROWS // _BLK
    return pl.pallas_call(
        _tc_scan_body,
        grid=(grid,),
        in_specs=[
            pl.BlockSpec((1, _KEY_BITS), lambda i: (0, 0)),
            pl.BlockSpec((_BLK, _KEY_BITS), lambda i: (i + sc_blocks, 0)),
        ],
        out_specs=pl.BlockSpec(memory_space=pltpu.SMEM),
        out_shape=jax.ShapeDtypeStruct((1,), jnp.int32),
        scratch_shapes=[pltpu.SMEM((1,), jnp.int32)],
    )(q2, keys)


def _merge_body(comb_sc_ref, comb_tc_ref, values_hbm, out_ref, sem):
    best = jnp.minimum(jnp.min(comb_sc_ref[...]), comb_tc_ref[0])
    idx = jnp.bitwise_and(best, _CAPACITY - 1)
    copy = pltpu.make_async_copy(values_hbm.at[idx], out_ref, sem)
    copy.start()
    copy.wait()


def _merge_gather(comb_sc, comb_tc, values):
    return pl.pallas_call(
        _merge_body,
        in_specs=[
            pl.BlockSpec(memory_space=pltpu.VMEM),
            pl.BlockSpec(memory_space=pltpu.SMEM),
            pl.BlockSpec(memory_space=pltpu.MemorySpace.HBM),
        ],
        out_specs=pl.BlockSpec(memory_space=pltpu.VMEM),
        out_shape=jax.ShapeDtypeStruct((_VALUE_BITS,), jnp.float32),
        scratch_shapes=[pltpu.SemaphoreType.DMA],
    )(comb_sc, comb_tc, values)


def kernel(query, keys, values):
    comb_sc = _sc_scan(query, keys)
    comb_tc = _tc_scan(query, keys)
    return _merge_gather(comb_sc, comb_tc, values)


# speedup vs baseline: 3.4855x; 1.4797x over previous
import jax
import jax.numpy as jnp
from jax import lax
from jax.experimental import pallas as pl
from jax.experimental.pallas import tpu as pltpu

_CAPACITY = 16384
_KEY_BITS = 2048
_VALUE_BITS = 2048
_BLK = 512
_NBLK = _CAPACITY // _BLK
_NBUF = 4


def _blk_start(keys_hbm, kbuf, sems, blk, b):
    copy = pltpu.make_async_copy(
        keys_hbm.at[pl.ds(blk * _BLK, _BLK)], kbuf.at[b], sems[b]
    )
    copy.start()


def _body(q_ref, keys_hbm, values_hbm, out_ref, kbuf, best_ref, gsem, *sems):
    for b in range(_NBUF):
        _blk_start(keys_hbm, kbuf, sems, b, b)
    best_ref[0] = jnp.int32(2**30)

    def super_body(s, _):
        for b in range(_NBUF):
            blk = s * _NBUF + b
            pltpu.make_async_copy(
                keys_hbm.at[pl.ds(blk * _BLK, _BLK)], kbuf.at[b], sems[b]
            ).wait()
            xor = jnp.bitwise_xor(kbuf[b], q_ref[...])
            dist = jnp.sum(xor, axis=1, keepdims=True)
            rows = lax.broadcasted_iota(jnp.int32, dist.shape, 0)
            combined = dist * _CAPACITY + (blk * _BLK + rows)
            best_ref[0] = jnp.minimum(best_ref[0], jnp.min(combined))

            @pl.when(blk + _NBUF < _NBLK)
            def _start_next():
                _blk_start(keys_hbm, kbuf, sems, blk + _NBUF, b)
        return 0

    lax.fori_loop(0, _NBLK // _NBUF, super_body, 0)

    idx = jnp.bitwise_and(best_ref[0], _CAPACITY - 1)
    copy = pltpu.make_async_copy(values_hbm.at[idx], out_ref, gsem)
    copy.start()
    copy.wait()


def kernel(query, keys, values):
    q2 = query.reshape(1, _KEY_BITS)
    return pl.pallas_call(
        _body,
        in_specs=[
            pl.BlockSpec(memory_space=pltpu.VMEM),
            pl.BlockSpec(memory_space=pltpu.MemorySpace.HBM),
            pl.BlockSpec(memory_space=pltpu.MemorySpace.HBM),
        ],
        out_specs=pl.BlockSpec(memory_space=pltpu.VMEM),
        out_shape=jax.ShapeDtypeStruct((_VALUE_BITS,), jnp.float32),
        scratch_shapes=[
            pltpu.VMEM((_NBUF, _BLK, _KEY_BITS), jnp.int32),
            pltpu.SMEM((1,), jnp.int32),
            pltpu.SemaphoreType.DMA,
        ]
        + [pltpu.SemaphoreType.DMA] * _NBUF,
    )(q2, keys, values)
